# Initial kernel scaffold; baseline (speedup 1.0000x reference)
import jax
import jax.numpy as jnp
from jax.experimental import pallas as pl


def kernel(x, edge_index):
    raise NotImplementedError("write your pallas kernel here")


# speedup vs baseline: 3.8820x; 3.8820x over previous
import functools

import jax
import jax.numpy as jnp
from jax import lax
from jax.experimental import pallas as pl
from jax.experimental.pallas import tpu as pltpu
from jax.experimental.pallas import tpu_sc as plsc

_E = 320000
_D = 128
_NC = 2
_NS = 16
_NW = _NC * _NS
_PER_W = _E // _NW
_K = 80
_NCHUNK = _PER_W // _K
_L = 16


def _dot_chunk(srows, drows, obuf):
    lane = jnp.arange(_L, dtype=jnp.int32)

    def gbody(g, carry):
        def ebody(i, vec):
            e = g * _L + i
            acc = srows[e, pl.ds(0, _L)] * drows[e, pl.ds(0, _L)]
            for j in range(1, _D // _L):
                acc = acc + (srows[e, pl.ds(j * _L, _L)] *
                             drows[e, pl.ds(j * _L, _L)])
            return jnp.where(lane == i, jnp.sum(acc), vec)

        vec = lax.fori_loop(0, _L, ebody, jnp.zeros((_L,), jnp.float32),
                            unroll=4)
        obuf[pl.ds(pl.multiple_of(g * _L, _L), _L)] = vec
        return carry

    lax.fori_loop(0, _K // _L, gbody, 0)


@functools.partial(
    pl.kernel,
    out_type=jax.ShapeDtypeStruct((_E,), jnp.float32),
    mesh=plsc.VectorSubcoreMesh(core_axis_name="c", subcore_axis_name="s"),
    compiler_params=pltpu.CompilerParams(needs_layout_passes=False),
    scratch_types=[
        pltpu.VMEM((_K,), jnp.int32),
        pltpu.VMEM((_K,), jnp.int32),
        pltpu.VMEM((_K, _D), jnp.float32),
        pltpu.VMEM((_K, _D), jnp.float32),
        pltpu.VMEM((_K,), jnp.float32),
        pltpu.SemaphoreType.DMA,
    ],
)
def _score_kernel(x_hbm, src_hbm, dst_hbm, out_hbm,
                  sidx, didx, srows, drows, obuf, sem):
    wid = lax.axis_index("s") * _NC + lax.axis_index("c")

    def chunk_body(c, carry):
        base = pl.multiple_of(wid * _PER_W + c * _K, _K)
        pltpu.sync_copy(src_hbm.at[pl.ds(base, _K)], sidx)
        pltpu.sync_copy(dst_hbm.at[pl.ds(base, _K)], didx)
        cp_s = pltpu.async_copy(x_hbm.at[sidx], srows, sem)
        cp_d = pltpu.async_copy(x_hbm.at[didx], drows, sem)
        cp_s.wait()
        cp_d.wait()
        _dot_chunk(srows, drows, obuf)
        pltpu.sync_copy(obuf, out_hbm.at[pl.ds(base, _K)])
        return carry

    lax.fori_loop(0, _NCHUNK, chunk_body, 0)


def kernel(x, edge_index):
    src = edge_index[0].astype(jnp.int32)
    dst = edge_index[1].astype(jnp.int32)
    out = _score_kernel(x, src, dst)
    return out.reshape(_E, 1)


# speedup vs baseline: 7.2145x; 1.8584x over previous
import functools

import jax
import jax.numpy as jnp
from jax import lax
from jax.experimental import pallas as pl
from jax.experimental.pallas import tpu as pltpu
from jax.experimental.pallas import tpu_sc as plsc

_E = 320000
_D = 128
_NC = 2
_NS = 16
_NW = _NC * _NS
_PER_W = _E // _NW
_K = 80
_NCHUNK = _PER_W // _K
_L = 16


def _dot_chunk(srows, drows, obuf):
    lane = jnp.arange(_L, dtype=jnp.int32)

    def gbody(g, carry):
        def ebody(i, vec):
            e = g * _L + i
            acc = srows[e, pl.ds(0, _L)] * drows[e, pl.ds(0, _L)]
            for j in range(1, _D // _L):
                acc = acc + (srows[e, pl.ds(j * _L, _L)] *
                             drows[e, pl.ds(j * _L, _L)])
            return jnp.where(lane == i, jnp.sum(acc), vec)

        vec = lax.fori_loop(0, _L, ebody, jnp.zeros((_L,), jnp.float32),
                            unroll=4)
        obuf[pl.ds(pl.multiple_of(g * _L, _L), _L)] = vec
        return carry

    lax.fori_loop(0, _K // _L, gbody, 0)


@functools.partial(
    pl.kernel,
    out_type=jax.ShapeDtypeStruct((_E,), jnp.float32),
    mesh=plsc.VectorSubcoreMesh(core_axis_name="c", subcore_axis_name="s"),
    compiler_params=pltpu.CompilerParams(needs_layout_passes=False),
    scratch_types=[
        pltpu.VMEM((2, _K), jnp.int32),
        pltpu.VMEM((2, _K), jnp.int32),
        pltpu.VMEM((2, _K, _D), jnp.float32),
        pltpu.VMEM((2, _K, _D), jnp.float32),
        pltpu.VMEM((2, _K), jnp.float32),
        pltpu.SemaphoreType.DMA,
        pltpu.SemaphoreType.DMA,
        pltpu.SemaphoreType.DMA,
        pltpu.SemaphoreType.DMA,
        pltpu.SemaphoreType.DMA,
        pltpu.SemaphoreType.DMA,
    ],
)
def _score_kernel(x_hbm, src_hbm, dst_hbm, out_hbm,
                  sidx, didx, srows, drows, obuf,
                  sem_i0, sem_i1, sem_g0, sem_g1, sem_o0, sem_o1):
    wid = lax.axis_index("s") * _NC + lax.axis_index("c")
    sem_i = (sem_i0, sem_i1)
    sem_g = (sem_g0, sem_g1)
    sem_o = (sem_o0, sem_o1)

    def issue_idx(c, b):
        base = pl.multiple_of(wid * _PER_W + c * _K, _K)
        pltpu.async_copy(src_hbm.at[pl.ds(base, _K)], sidx.at[b], sem_i[b])
        pltpu.async_copy(dst_hbm.at[pl.ds(base, _K)], didx.at[b], sem_i[b])

    def wait_idx(b):
        pltpu.make_async_copy(src_hbm.at[pl.ds(0, _K)], sidx.at[b],
                              sem_i[b]).wait()
        pltpu.make_async_copy(dst_hbm.at[pl.ds(0, _K)], didx.at[b],
                              sem_i[b]).wait()

    def issue_gather(b):
        pltpu.async_copy(x_hbm.at[sidx.at[b]], srows.at[b], sem_g[b])
        pltpu.async_copy(x_hbm.at[didx.at[b]], drows.at[b], sem_g[b])

    def wait_gather(b):
        pltpu.make_async_copy(x_hbm.at[pl.ds(0, _K)], srows.at[b],
                              sem_g[b]).wait()
        pltpu.make_async_copy(x_hbm.at[pl.ds(0, _K)], drows.at[b],
                              sem_g[b]).wait()

    def wait_out(b):
        pltpu.make_async_copy(obuf.at[b], out_hbm.at[pl.ds(0, _K)],
                              sem_o[b]).wait()

    def step(c, b):
        nb = 1 - b
        wait_gather(b)

        @pl.when(c + 1 < _NCHUNK)
        def _():
            wait_idx(nb)
            issue_gather(nb)

        @pl.when(c + 2 < _NCHUNK)
        def _():
            issue_idx(c + 2, b)

        @pl.when(c >= 2)
        def _():
            wait_out(b)

        _dot_chunk(srows.at[b], drows.at[b], obuf.at[b])
        base = pl.multiple_of(wid * _PER_W + c * _K, _K)
        pltpu.async_copy(obuf.at[b], out_hbm.at[pl.ds(base, _K)], sem_o[b])

    issue_idx(0, 0)
    wait_idx(0)
    issue_gather(0)
    issue_idx(1, 1)

    def pair(j, carry):
        step(j * 2, 0)
        step(j * 2 + 1, 1)
        return carry

    lax.fori_loop(0, _NCHUNK // 2, pair, 0)
    step(_NCHUNK - 1, 0)
    wait_out(0)
    wait_out(1)


def kernel(x, edge_index):
    src = edge_index[0].astype(jnp.int32)
    dst = edge_index[1].astype(jnp.int32)
    out = _score_kernel(x, src, dst)
    return out.reshape(_E, 1)


# speedup vs baseline: 7.2349x; 1.0028x over previous
import functools

import jax
import jax.numpy as jnp
from jax import lax
from jax.experimental import pallas as pl
from jax.experimental.pallas import tpu as pltpu
from jax.experimental.pallas import tpu_sc as plsc

_E = 320000
_D = 128
_NC = 2
_NS = 16
_NW = _NC * _NS
_PER_W = _E // _NW
_K = 80
_NCHUNK = _PER_W // _K
_L = 16


def _dot_chunk(srows, drows, obuf):
    lane = jnp.arange(_L, dtype=jnp.int32)

    def gbody(g, carry):
        def ebody(i, vec):
            e = g * _L + i
            acc0 = jnp.zeros((_L,), jnp.float32)
            acc1 = jnp.zeros((_L,), jnp.float32)
            for j in range(_D // (2 * _L)):
                acc0 = acc0 + (srows[e, pl.ds(2 * j * _L, _L)] *
                               drows[e, pl.ds(2 * j * _L, _L)])
                acc1 = acc1 + (srows[e, pl.ds((2 * j + 1) * _L, _L)] *
                               drows[e, pl.ds((2 * j + 1) * _L, _L)])
            return jnp.where(lane == i, jnp.sum(acc0 + acc1), vec)

        vec = lax.fori_loop(0, _L, ebody, jnp.zeros((_L,), jnp.float32),
                            unroll=4)
        obuf[pl.ds(pl.multiple_of(g * _L, _L), _L)] = vec
        return carry

    lax.fori_loop(0, _K // _L, gbody, 0)


@functools.partial(
    pl.kernel,
    out_type=jax.ShapeDtypeStruct((_E,), jnp.float32),
    mesh=plsc.VectorSubcoreMesh(core_axis_name="c", subcore_axis_name="s"),
    compiler_params=pltpu.CompilerParams(needs_layout_passes=False),
    scratch_types=[
        pltpu.VMEM((2, _K), jnp.int32),
        pltpu.VMEM((2, _K), jnp.int32),
        pltpu.VMEM((2, _K, _D), jnp.float32),
        pltpu.VMEM((2, _K, _D), jnp.float32),
        pltpu.VMEM((2, _K), jnp.float32),
        pltpu.SemaphoreType.DMA,
        pltpu.SemaphoreType.DMA,
        pltpu.SemaphoreType.DMA,
        pltpu.SemaphoreType.DMA,
        pltpu.SemaphoreType.DMA,
        pltpu.SemaphoreType.DMA,
    ],
)
def _score_kernel(x_hbm, src_hbm, dst_hbm, out_hbm,
                  sidx, didx, srows, drows, obuf,
                  sem_i0, sem_i1, sem_g0, sem_g1, sem_o0, sem_o1):
    wid = lax.axis_index("s") * _NC + lax.axis_index("c")
    sem_i = (sem_i0, sem_i1)
    sem_g = (sem_g0, sem_g1)
    sem_o = (sem_o0, sem_o1)

    def issue_idx(c, b):
        base = pl.multiple_of(wid * _PER_W + c * _K, _K)
        pltpu.async_copy(src_hbm.at[pl.ds(base, _K)], sidx.at[b], sem_i[b])
        pltpu.async_copy(dst_hbm.at[pl.ds(base, _K)], didx.at[b], sem_i[b])

    def wait_idx(b):
        pltpu.make_async_copy(src_hbm.at[pl.ds(0, _K)], sidx.at[b],
                              sem_i[b]).wait()
        pltpu.make_async_copy(dst_hbm.at[pl.ds(0, _K)], didx.at[b],
                              sem_i[b]).wait()

    def issue_gather(b):
        pltpu.async_copy(x_hbm.at[sidx.at[b]], srows.at[b], sem_g[b])
        pltpu.async_copy(x_hbm.at[didx.at[b]], drows.at[b], sem_g[b])

    def wait_gather(b):
        pltpu.make_async_copy(x_hbm.at[pl.ds(0, _K)], srows.at[b],
                              sem_g[b]).wait()
        pltpu.make_async_copy(x_hbm.at[pl.ds(0, _K)], drows.at[b],
                              sem_g[b]).wait()

    def wait_out(b):
        pltpu.make_async_copy(obuf.at[b], out_hbm.at[pl.ds(0, _K)],
                              sem_o[b]).wait()

    def step(c, b):
        nb = 1 - b
        wait_gather(b)

        @pl.when(c + 1 < _NCHUNK)
        def _():
            wait_idx(nb)
            issue_gather(nb)

        @pl.when(c + 2 < _NCHUNK)
        def _():
            issue_idx(c + 2, b)

        @pl.when(c >= 2)
        def _():
            wait_out(b)

        _dot_chunk(srows.at[b], drows.at[b], obuf.at[b])
        base = pl.multiple_of(wid * _PER_W + c * _K, _K)
        pltpu.async_copy(obuf.at[b], out_hbm.at[pl.ds(base, _K)], sem_o[b])

    issue_idx(0, 0)
    wait_idx(0)
    issue_gather(0)
    issue_idx(1, 1)

    def pair(j, carry):
        step(j * 2, 0)
        step(j * 2 + 1, 1)
        return carry

    lax.fori_loop(0, _NCHUNK // 2, pair, 0)
    step(_NCHUNK - 1, 0)
    wait_out(0)
    wait_out(1)


def kernel(x, edge_index):
    src = edge_index[0].astype(jnp.int32)
    dst = edge_index[1].astype(jnp.int32)
    out = _score_kernel(x, src, dst)
    return out.reshape(_E, 1)


# speedup vs baseline: 8.7905x; 1.2150x over previous
import functools

import jax
import jax.numpy as jnp
from jax import lax
from jax.experimental import pallas as pl
from jax.experimental.pallas import tpu as pltpu
from jax.experimental.pallas import tpu_sc as plsc

_E = 320000
_D = 128
_NC = 2
_NS = 16
_NW = _NC * _NS
_PER_W = _E // _NW
_K = 80
_NCHUNK = _PER_W // _K
_L = 16


def _dot_chunk(srows, drows, obuf):
    lane = jnp.arange(_L, dtype=jnp.int32)

    def gbody(g, carry):
        def ebody(i, vec):
            e = g * _L + i
            acc0 = jnp.zeros((_L,), jnp.float32)
            acc1 = jnp.zeros((_L,), jnp.float32)
            for j in range(_D // (2 * _L)):
                acc0 = acc0 + (srows[e, pl.ds(2 * j * _L, _L)] *
                               drows[e, pl.ds(2 * j * _L, _L)])
                acc1 = acc1 + (srows[e, pl.ds((2 * j + 1) * _L, _L)] *
                               drows[e, pl.ds((2 * j + 1) * _L, _L)])
            return jnp.where(lane == i, jnp.sum(acc0 + acc1), vec)

        vec = lax.fori_loop(0, _L, ebody, jnp.zeros((_L,), jnp.float32),
                            unroll=4)
        obuf[pl.ds(pl.multiple_of(g * _L, _L), _L)] = vec
        return carry

    lax.fori_loop(0, _K // _L, gbody, 0)


@functools.partial(
    pl.kernel,
    out_type=jax.ShapeDtypeStruct((_E,), jnp.float32),
    mesh=plsc.VectorSubcoreMesh(core_axis_name="c", subcore_axis_name="s"),
    compiler_params=pltpu.CompilerParams(needs_layout_passes=False),
    scratch_types=[
        pltpu.VMEM((2, _K), jnp.int32),
        pltpu.VMEM((2, _K), jnp.int32),
        pltpu.VMEM((2, _K, _D), jnp.float32),
        pltpu.VMEM((2, _K, _D), jnp.float32),
        pltpu.VMEM((2, _K), jnp.float32),
        pltpu.SemaphoreType.DMA,
        pltpu.SemaphoreType.DMA,
        pltpu.SemaphoreType.DMA,
        pltpu.SemaphoreType.DMA,
        pltpu.SemaphoreType.DMA,
        pltpu.SemaphoreType.DMA,
    ],
)
def _score_kernel(x_hbm, src_hbm, dst_hbm, out_hbm,
                  sidx, didx, srows, drows, obuf,
                  sem_i0, sem_i1, sem_g0, sem_g1, sem_o0, sem_o1):
    wid = lax.axis_index("s") * _NC + lax.axis_index("c")
    sem_i = (sem_i0, sem_i1)
    sem_g = (sem_g0, sem_g1)
    sem_o = (sem_o0, sem_o1)

    def issue_idx(c, b):
        base = pl.multiple_of(wid * _PER_W + c * _K, _K)
        pltpu.async_copy(src_hbm.at[pl.ds(base, _K)], sidx.at[b], sem_i[b])
        pltpu.async_copy(dst_hbm.at[pl.ds(base, _K)], didx.at[b], sem_i[b])

    def wait_idx(b):
        pltpu.make_async_copy(src_hbm.at[pl.ds(0, _K)], sidx.at[b],
                              sem_i[b]).wait()
        pltpu.make_async_copy(dst_hbm.at[pl.ds(0, _K)], didx.at[b],
                              sem_i[b]).wait()

    def issue_gather(b):
        pltpu.async_copy(x_hbm.at[sidx.at[b]], srows.at[b], sem_g[b])
        pltpu.async_copy(x_hbm.at[didx.at[b]], drows.at[b], sem_g[b])

    def wait_gather(b):
        pltpu.make_async_copy(x_hbm.at[pl.ds(0, _K)], srows.at[b],
                              sem_g[b]).wait()
        pltpu.make_async_copy(x_hbm.at[pl.ds(0, _K)], drows.at[b],
                              sem_g[b]).wait()

    def wait_out(b):
        pltpu.make_async_copy(obuf.at[b], out_hbm.at[pl.ds(0, _K)],
                              sem_o[b]).wait()

    def step(c, b):
        nb = 1 - b

        @pl.when(c + 1 < _NCHUNK)
        def _():
            wait_idx(nb)
            issue_gather(nb)

        wait_gather(b)

        @pl.when(c + 2 < _NCHUNK)
        def _():
            issue_idx(c + 2, b)

        @pl.when(c >= 2)
        def _():
            wait_out(b)

        _dot_chunk(srows.at[b], drows.at[b], obuf.at[b])
        base = pl.multiple_of(wid * _PER_W + c * _K, _K)
        pltpu.async_copy(obuf.at[b], out_hbm.at[pl.ds(base, _K)], sem_o[b])

    issue_idx(0, 0)
    wait_idx(0)
    issue_gather(0)
    issue_idx(1, 1)

    def pair(j, carry):
        step(j * 2, 0)
        step(j * 2 + 1, 1)
        return carry

    lax.fori_loop(0, _NCHUNK // 2, pair, 0)
    step(_NCHUNK - 1, 0)
    wait_out(0)
    wait_out(1)


def kernel(x, edge_index):
    src = edge_index[0].astype(jnp.int32)
    dst = edge_index[1].astype(jnp.int32)
    out = _score_kernel(x, src, dst)
    return out.reshape(_E, 1)


# speedup vs baseline: 10.4542x; 1.1893x over previous
import functools

import jax
import jax.numpy as jnp
from jax import lax
from jax.experimental import pallas as pl
from jax.experimental.pallas import tpu as pltpu
from jax.experimental.pallas import tpu_sc as plsc

_E = 320000
_D = 128
_NC = 2
_NS = 16
_NW = _NC * _NS
_PER_W = _E // _NW
_K = 80
_NCHUNK = _PER_W // _K
_L = 16


def _dot_chunk(srows, drows, obuf):
    lane = jnp.arange(_L, dtype=jnp.int32)

    def gbody(g, carry):
        def ebody(i, vec):
            e = g * _L + i
            acc0 = jnp.zeros((_L,), jnp.float32)
            acc1 = jnp.zeros((_L,), jnp.float32)
            for j in range(_D // (2 * _L)):
                acc0 = acc0 + (srows[e, pl.ds(2 * j * _L, _L)] *
                               drows[e, pl.ds(2 * j * _L, _L)])
                acc1 = acc1 + (srows[e, pl.ds((2 * j + 1) * _L, _L)] *
                               drows[e, pl.ds((2 * j + 1) * _L, _L)])
            return jnp.where(lane == i, jnp.sum(acc0 + acc1), vec)

        vec = lax.fori_loop(0, _L, ebody, jnp.zeros((_L,), jnp.float32),
                            unroll=4)
        obuf[pl.ds(pl.multiple_of(g * _L, _L), _L)] = vec
        return carry

    lax.fori_loop(0, _K // _L, gbody, 0)


_NBUF = 4

@functools.partial(
    pl.kernel,
    out_type=jax.ShapeDtypeStruct((_E,), jnp.float32),
    mesh=plsc.VectorSubcoreMesh(core_axis_name="c", subcore_axis_name="s"),
    compiler_params=pltpu.CompilerParams(needs_layout_passes=False),
    scratch_types=[
        pltpu.VMEM((_NBUF, _K), jnp.int32),
        pltpu.VMEM((_NBUF, _K), jnp.int32),
        pltpu.VMEM((_NBUF, _K, _D), jnp.float32),
        pltpu.VMEM((_NBUF, _K, _D), jnp.float32),
        pltpu.VMEM((_NBUF, _K), jnp.float32),
    ] + [pltpu.SemaphoreType.DMA] * (3 * _NBUF),
)
def _score_kernel(x_hbm, src_hbm, dst_hbm, out_hbm,
                  sidx, didx, srows, drows, obuf, *sems):
    wid = lax.axis_index("s") * _NC + lax.axis_index("c")
    sem_i = sems[0:_NBUF]
    sem_g = sems[_NBUF:2 * _NBUF]
    sem_o = sems[2 * _NBUF:3 * _NBUF]

    def issue_idx(c, b):
        base = pl.multiple_of(wid * _PER_W + c * _K, _K)
        pltpu.async_copy(src_hbm.at[pl.ds(base, _K)], sidx.at[b], sem_i[b])
        pltpu.async_copy(dst_hbm.at[pl.ds(base, _K)], didx.at[b], sem_i[b])

    def wait_idx(b):
        pltpu.make_async_copy(src_hbm.at[pl.ds(0, _K)], sidx.at[b],
                              sem_i[b]).wait()
        pltpu.make_async_copy(dst_hbm.at[pl.ds(0, _K)], didx.at[b],
                              sem_i[b]).wait()

    def issue_gather(b):
        pltpu.async_copy(x_hbm.at[sidx.at[b]], srows.at[b], sem_g[b])
        pltpu.async_copy(x_hbm.at[didx.at[b]], drows.at[b], sem_g[b])

    def wait_gather(b):
        pltpu.make_async_copy(x_hbm.at[pl.ds(0, _K)], srows.at[b],
                              sem_g[b]).wait()
        pltpu.make_async_copy(x_hbm.at[pl.ds(0, _K)], drows.at[b],
                              sem_g[b]).wait()

    def wait_out(b):
        pltpu.make_async_copy(obuf.at[b], out_hbm.at[pl.ds(0, _K)],
                              sem_o[b]).wait()

    def step(c, b):
        gb = (b + _NBUF - 1) % _NBUF

        @pl.when(c + _NBUF - 1 < _NCHUNK)
        def _():
            wait_idx(gb)
            issue_gather(gb)

        wait_gather(b)

        @pl.when(c + _NBUF < _NCHUNK)
        def _():
            issue_idx(c + _NBUF, b)

        @pl.when(c >= _NBUF)
        def _():
            wait_out(b)

        _dot_chunk(srows.at[b], drows.at[b], obuf.at[b])
        base = pl.multiple_of(wid * _PER_W + c * _K, _K)
        pltpu.async_copy(obuf.at[b], out_hbm.at[pl.ds(base, _K)], sem_o[b])

    for c in range(_NBUF):
        issue_idx(c, c)
    for c in range(_NBUF - 1):
        wait_idx(c)
        issue_gather(c)

    def quad(j, carry):
        for b in range(_NBUF):
            step(j * _NBUF + b, b)
        return carry

    lax.fori_loop(0, _NCHUNK // _NBUF, quad, 0)
    step(_NCHUNK - 1, (_NCHUNK - 1) % _NBUF)
    for b in range(_NBUF):
        wait_out(b)


def kernel(x, edge_index):
    src = edge_index[0].astype(jnp.int32)
    dst = edge_index[1].astype(jnp.int32)
    out = _score_kernel(x, src, dst)
    return out.reshape(_E, 1)
